# baseline (speedup 1.0000x reference)
import jax
import jax.numpy as jnp
from jax.experimental import pallas as pl
from jax.experimental.pallas import tpu as pltpu

B = 16
S = 2048
F = 32
H = 64
SE = S - 1
NT = SE - H + 1
NC = 16
NQ = NT // NC
DEC_F = 8
DEC_W = H * DEC_F


def _fork_body(x_ref, m_ref, xr_ref, mr_ref,
               enc_ref, dec_ref, his_ref, fut_ref):
    r = pl.program_id(1)

    @pl.when(r == 0)
    def _():
        enc_ref[0] = x_ref[0, :SE, :24]
        his_ref[0] = m_ref[0, :SE, :]

    A = xr_ref[0]
    sh = 120 - 8 * r
    lane = jax.lax.broadcasted_iota(jnp.int32, (NQ, 128), 1)
    keep = lane < sh
    pieces = []
    for k in range(4):
        r1 = pltpu.roll(A[k:k + NQ], sh, axis=1)
        r2 = pltpu.roll(A[k + 1:k + 1 + NQ], sh, axis=1)
        pieces.append(jnp.where(keep, r1, r2))
    dec_ref[0, :, 0, 0, :] = jnp.concatenate(pieces, axis=1)

    M0 = mr_ref[0]
    M1 = jnp.concatenate([M0[1:], M0[:1]], axis=0)
    R0 = jnp.broadcast_to(M0[:, None, :], (16, 8, 128)).reshape(128, 128)
    R1 = jnp.broadcast_to(M1[:, None, :], (16, 8, 128)).reshape(128, 128)
    c = r + 1
    R0s = pltpu.roll(R0, 0, axis=1, stride=16, stride_axis=0)
    R1s = pltpu.roll(R1, 0, axis=1, stride=16, stride_axis=0)
    F0 = pltpu.roll(R0s, 16 - c, axis=1)
    F1 = pltpu.roll(R1s, 16 - c, axis=1)
    lane8 = jax.lax.broadcasted_iota(jnp.int32, (128, 128), 1)
    subu = jax.lax.broadcasted_iota(jnp.int32, (128, 128), 0) % 8
    Vr = jnp.where(lane8 + 112 - 16 * subu + c < 128, F0, F1)
    Vr3 = Vr.reshape(16, 8, 128)
    V = jnp.concatenate([Vr3[:, 7 - i:8 - i, :] for i in range(8)],
                        axis=1).reshape(128, 128)
    fut_ref[0, :, 0, 0, :] = V[:NQ, :H]


def kernel(tensor, masking):
    xr = tensor[:, :, 24:32].reshape(B, 128, 128)
    mr = masking.reshape(B, NC, 128)
    enc, dec5, his, fut5 = pl.pallas_call(
        _fork_body,
        grid=(B, NC),
        in_specs=[
            pl.BlockSpec((1, S, F), lambda b, r: (b, 0, 0)),
            pl.BlockSpec((1, S, 1), lambda b, r: (b, 0, 0)),
            pl.BlockSpec((1, 128, 128), lambda b, r: (b, 0, 0)),
            pl.BlockSpec((1, NC, 128), lambda b, r: (b, 0, 0)),
        ],
        out_specs=[
            pl.BlockSpec((1, SE, 24), lambda b, r: (b, 0, 0)),
            pl.BlockSpec((1, NQ, 1, 1, DEC_W), lambda b, r: (b, 0, r, 0, 0)),
            pl.BlockSpec((1, SE, 1), lambda b, r: (b, 0, 0)),
            pl.BlockSpec((1, NQ, 1, 1, H), lambda b, r: (b, 0, r, 0, 0)),
        ],
        out_shape=[
            jax.ShapeDtypeStruct((B, SE, 24), jnp.float32),
            jax.ShapeDtypeStruct((B, NQ, NC, 1, DEC_W), jnp.float32),
            jax.ShapeDtypeStruct((B, SE, 1), jnp.float32),
            jax.ShapeDtypeStruct((B, NQ, NC, 1, H), jnp.float32),
        ],
        compiler_params=pltpu.CompilerParams(
            dimension_semantics=("parallel", "arbitrary"),
        ),
    )(tensor, masking, xr, mr)
    dec = dec5.reshape(B, NT, H, DEC_F)
    fut = fut5.reshape(B, NT, H, 1)
    return (enc, dec, his, fut)


# speedup vs baseline: 1.1092x; 1.1092x over previous
import jax
import jax.numpy as jnp
from jax.experimental import pallas as pl
from jax.experimental.pallas import tpu as pltpu

B = 16
S = 2048
F = 32
H = 64
SE = S - 1
NT = SE - H + 1
NC = 16
NQ = NT // NC
DEC_F = 8
DEC_W = H * DEC_F


def _fork_body(x_ref, m_ref, xr_ref, mr_ref,
               enc_ref, dec_ref, his_ref, fut_ref):
    enc_ref[0] = x_ref[0, :SE, :24]
    his_ref[0] = m_ref[0, :SE, :]

    A = xr_ref[0]
    for r in range(NC):
        s = 8 * r + 8
        pieces = []
        for k in range(4):
            if s == 128:
                pieces.append(A[k + 1:k + 1 + NQ])
            else:
                a1 = A[k:k + NQ]
                a2 = A[k + 1:k + 1 + NQ]
                pieces.append(
                    jnp.concatenate([a1[:, s:], a2[:, :s]], axis=1))
        dec_ref[0, :, r, :] = jnp.concatenate(pieces, axis=1)

    Mv = mr_ref[0]
    lane = jax.lax.broadcasted_iota(jnp.int32, (8, 128), 1)
    subu = jax.lax.broadcasted_iota(jnp.int32, (8, 128), 0)
    for m in range(NT // 8):
        st = 8 * m + 1
        p, v0 = divmod(st, 128)
        sh = (128 - v0 - 7) % 128
        b1 = jnp.broadcast_to(Mv[p:p + 1], (8, 128))
        r1 = pltpu.roll(b1, sh, axis=1, stride=1, stride_axis=0)
        if v0 + 70 >= 128:
            b2 = jnp.broadcast_to(Mv[p + 1:p + 2], (8, 128))
            r2 = pltpu.roll(b2, sh, axis=1, stride=1, stride_axis=0)
            piece = jnp.where(lane + 7 - subu < 128 - v0, r1, r2)
        else:
            piece = r1
        for u in range(8):
            t = 8 * m + 7 - u
            fut_ref[0, t:t + 1, :] = piece[u:u + 1, :H]


def kernel(tensor, masking):
    xr = tensor[:, :, 24:32].reshape(B, 128, 128)
    mr = masking.reshape(B, NC, 128)
    enc, dec4, his, fut = pl.pallas_call(
        _fork_body,
        grid=(B,),
        in_specs=[
            pl.BlockSpec((1, S, F), lambda b: (b, 0, 0)),
            pl.BlockSpec((1, S, 1), lambda b: (b, 0, 0)),
            pl.BlockSpec((1, 128, 128), lambda b: (b, 0, 0)),
            pl.BlockSpec((1, NC, 128), lambda b: (b, 0, 0)),
        ],
        out_specs=[
            pl.BlockSpec((1, SE, 24), lambda b: (b, 0, 0)),
            pl.BlockSpec((1, NQ, NC, DEC_W), lambda b: (b, 0, 0, 0)),
            pl.BlockSpec((1, SE, 1), lambda b: (b, 0, 0)),
            pl.BlockSpec((1, NT, H), lambda b: (b, 0, 0)),
        ],
        out_shape=[
            jax.ShapeDtypeStruct((B, SE, 24), jnp.float32),
            jax.ShapeDtypeStruct((B, NQ, NC, DEC_W), jnp.float32),
            jax.ShapeDtypeStruct((B, SE, 1), jnp.float32),
            jax.ShapeDtypeStruct((B, NT, H), jnp.float32),
        ],
        compiler_params=pltpu.CompilerParams(
            dimension_semantics=("parallel",),
        ),
    )(tensor, masking, xr, mr)
    dec = dec4.reshape(B, NT, H, DEC_F)
    fut = fut.reshape(B, NT, H, 1)
    return (enc, dec, his, fut)


# speedup vs baseline: 5.5246x; 4.9807x over previous
import jax
import jax.numpy as jnp
from jax.experimental import pallas as pl
from jax.experimental.pallas import tpu as pltpu

B = 16
S = 2048
F = 32
H = 64
SE = S - 1
NT = SE - H + 1
NC = 16
NQ = NT // NC
DEC_F = 8
DEC_W = H * DEC_F


def _fork_body(x_ref, m_ref, xr_ref, mr_ref,
               enc_ref, dec_ref, his_ref, fut_ref):
    enc_ref[0] = x_ref[0, :SE, :24]
    his_ref[0] = m_ref[0, :SE, :]

    A = xr_ref[0]
    for r in range(NC):
        s = 8 * r + 8
        pieces = []
        for k in range(4):
            if s == 128:
                pieces.append(A[k + 1:k + 1 + NQ])
            else:
                a1 = A[k:k + NQ]
                a2 = A[k + 1:k + 1 + NQ]
                pieces.append(
                    jnp.concatenate([a1[:, s:], a2[:, :s]], axis=1))
        cls = jnp.concatenate(pieces, axis=1)
        for q in range(NQ):
            t = NC * q + r
            dec_ref[0, t:t + 1, :] = cls[q:q + 1]

    Mv = mr_ref[0]
    lane = jax.lax.broadcasted_iota(jnp.int32, (8, 128), 1)
    subu = jax.lax.broadcasted_iota(jnp.int32, (8, 128), 0)
    for m in range(NT // 8):
        st = 8 * m + 1
        p, v0 = divmod(st, 128)
        sh = (128 - v0 - 7) % 128
        b1 = jnp.broadcast_to(Mv[p:p + 1], (8, 128))
        r1 = pltpu.roll(b1, sh, axis=1, stride=1, stride_axis=0)
        if v0 + 70 >= 128:
            b2 = jnp.broadcast_to(Mv[p + 1:p + 2], (8, 128))
            r2 = pltpu.roll(b2, sh, axis=1, stride=1, stride_axis=0)
            piece = jnp.where(lane + 7 - subu < 128 - v0, r1, r2)
        else:
            piece = r1
        for u in range(8):
            t = 8 * m + 7 - u
            fut_ref[0, t:t + 1, :] = piece[u:u + 1, :H]


def kernel(tensor, masking):
    xr = tensor[:, :, 24:32].reshape(B, 128, 128)
    mr = masking.reshape(B, NC, 128)
    enc, dec4, his, fut = pl.pallas_call(
        _fork_body,
        grid=(B,),
        in_specs=[
            pl.BlockSpec((1, S, F), lambda b: (b, 0, 0)),
            pl.BlockSpec((1, S, 1), lambda b: (b, 0, 0)),
            pl.BlockSpec((1, 128, 128), lambda b: (b, 0, 0)),
            pl.BlockSpec((1, NC, 128), lambda b: (b, 0, 0)),
        ],
        out_specs=[
            pl.BlockSpec((1, SE, 24), lambda b: (b, 0, 0)),
            pl.BlockSpec((1, NT, DEC_W), lambda b: (b, 0, 0)),
            pl.BlockSpec((1, SE, 1), lambda b: (b, 0, 0)),
            pl.BlockSpec((1, NT, H), lambda b: (b, 0, 0)),
        ],
        out_shape=[
            jax.ShapeDtypeStruct((B, SE, 24), jnp.float32),
            jax.ShapeDtypeStruct((B, NT, DEC_W), jnp.float32),
            jax.ShapeDtypeStruct((B, SE, 1), jnp.float32),
            jax.ShapeDtypeStruct((B, NT, H), jnp.float32),
        ],
        compiler_params=pltpu.CompilerParams(
            dimension_semantics=("parallel",),
        ),
    )(tensor, masking, xr, mr)
    dec = dec4.reshape(B, NT, H, DEC_F)
    fut = fut.reshape(B, NT, H, 1)
    return (enc, dec, his, fut)
